# baseline (speedup 1.0000x reference)
import functools

import jax
import jax.numpy as jnp
from jax import lax
from jax.experimental import pallas as pl
from jax.experimental.pallas import tpu as pltpu
from jax.experimental.pallas import tpu_sc as plsc

EMB_DIM = 128
CHUNK = 256
LANES = 16


@functools.lru_cache(maxsize=None)
def _make_lookup(n_idx: int, n_emb: int, d: int):
    info = plsc.get_sparse_core_info()
    nw = info.num_cores * info.num_subcores
    assert n_idx % (nw * 2 * CHUNK) == 0
    per_w = n_idx // nw
    n_chunks = per_w // CHUNK
    mesh = plsc.VectorSubcoreMesh(core_axis_name="c", subcore_axis_name="s")

    @functools.partial(
        pl.kernel,
        mesh=mesh,
        compiler_params=pltpu.CompilerParams(needs_layout_passes=False),
        out_type=jax.ShapeDtypeStruct((n_idx * d,), jnp.float32),
        scratch_types=[
            pltpu.VMEM((n_emb * d,), jnp.float32),
            pltpu.VMEM((CHUNK,), jnp.int32),
            pltpu.VMEM((CHUNK,), jnp.int32),
            pltpu.VMEM((CHUNK * d,), jnp.float32),
            pltpu.VMEM((CHUNK * d,), jnp.float32),
            pltpu.SemaphoreType.DMA,
            pltpu.SemaphoreType.DMA,
        ],
    )
    def lookup(table_hbm, idx_hbm, out_hbm, table_v,
               idx0, idx1, rows0, rows1, o0, o1):
        wid = lax.axis_index("s") * info.num_cores + lax.axis_index("c")
        base = wid * per_w
        idx_bufs = (idx0, idx1)
        rows_bufs = (rows0, rows1)
        osem = (o0, o1)

        pltpu.sync_copy(table_hbm, table_v)

        lane_iota = lax.iota(jnp.int32, LANES)

        def assemble(i, b):
            pltpu.sync_copy(idx_hbm.at[pl.ds(base + i * CHUNK, CHUNK)],
                            idx_bufs[b])

            def block(r, carry):
                v = idx_bufs[b][pl.ds(r * LANES, LANES)]
                vd = v * d
                sbase = (r * LANES + lane_iota) * d
                for c in range(d):
                    vals = plsc.load_gather(table_v, [vd + c])
                    plsc.store_scatter(rows_bufs[b], [sbase + c], vals)
                return carry

            lax.fori_loop(0, CHUNK // LANES, block, 0)

        def start_store(i, b):
            pltpu.async_copy(rows_bufs[b],
                             out_hbm.at[pl.ds((base + i * CHUNK) * d, CHUNK * d)],
                             osem[b])

        def wait_store(i, b):
            pltpu.make_async_copy(rows_bufs[b],
                                  out_hbm.at[pl.ds((base + i * CHUNK) * d, CHUNK * d)],
                                  osem[b]).wait()

        assemble(0, 0)
        start_store(0, 0)
        assemble(1, 1)
        start_store(1, 1)

        def body(j, carry):
            i0 = 2 * j
            i1 = i0 + 1
            wait_store(i0 - 2, 0)
            assemble(i0, 0)
            start_store(i0, 0)
            wait_store(i1 - 2, 1)
            assemble(i1, 1)
            start_store(i1, 1)
            return carry

        lax.fori_loop(1, n_chunks // 2, body, 0)

        wait_store(n_chunks - 2, 0)
        wait_store(n_chunks - 1, 1)

    return lookup


def kernel(idx, x, table):
    del x
    b, l = idx.shape
    n = b * l
    d = table.shape[1]
    idx_flat = idx.reshape(n).astype(jnp.int32)
    table_flat = table.astype(jnp.float32).reshape(-1)
    lookup = _make_lookup(n, table.shape[0], d)
    out = lookup(table_flat, idx_flat)
    return out.reshape(b, l, d)


# speedup vs baseline: 4.8011x; 4.8011x over previous
import functools

import jax
import jax.numpy as jnp
from jax import lax
from jax.experimental import pallas as pl
from jax.experimental.pallas import tpu as pltpu
from jax.experimental.pallas import tpu_sc as plsc

EMB_DIM = 128
CHUNK = 256
LANES = 16


@functools.lru_cache(maxsize=None)
def _make_lookup(n_idx: int, n_emb: int, d: int):
    info = plsc.get_sparse_core_info()
    nw = info.num_cores * info.num_subcores
    assert n_idx % (nw * 2 * CHUNK) == 0
    per_w = n_idx // nw
    n_chunks = per_w // CHUNK
    mesh = plsc.VectorSubcoreMesh(core_axis_name="c", subcore_axis_name="s")

    @functools.partial(
        pl.kernel,
        mesh=mesh,
        compiler_params=pltpu.CompilerParams(needs_layout_passes=False),
        out_type=jax.ShapeDtypeStruct((n_idx * d,), jnp.float32),
        scratch_types=[
            pltpu.VMEM((n_emb * d,), jnp.float32),
            pltpu.VMEM((CHUNK,), jnp.int32),
            pltpu.VMEM((CHUNK,), jnp.int32),
            pltpu.VMEM((CHUNK * d,), jnp.float32),
            pltpu.VMEM((CHUNK * d,), jnp.float32),
            pltpu.SemaphoreType.DMA,
            pltpu.SemaphoreType.DMA,
        ],
    )
    def lookup(table_hbm, idx_hbm, out_hbm, table_v,
               idx0, idx1, rows0, rows1, o0, o1):
        wid = lax.axis_index("s") * info.num_cores + lax.axis_index("c")
        base = wid * per_w
        idx_bufs = (idx0, idx1)
        rows_bufs = (rows0, rows1)
        osem = (o0, o1)

        pltpu.sync_copy(table_hbm, table_v)

        lane_iota = lax.iota(jnp.int32, LANES)

        def assemble(i, b):
            pltpu.sync_copy(idx_hbm.at[pl.ds(base + i * CHUNK, CHUNK)],
                            idx_bufs[b])

            def block(r, carry):
                v = idx_bufs[b][pl.ds(r * LANES, LANES)]
                vd = v * d
                sbase = (r * LANES + lane_iota) * d
                for c in range(d):
                    cv = (lane_iota + c) & (d - 1)
                    vals = plsc.load_gather(table_v, [vd + cv])
                    plsc.store_scatter(rows_bufs[b], [sbase + cv], vals)
                return carry

            lax.fori_loop(0, CHUNK // LANES, block, 0)

        def start_store(i, b):
            pltpu.async_copy(rows_bufs[b],
                             out_hbm.at[pl.ds((base + i * CHUNK) * d, CHUNK * d)],
                             osem[b])

        def wait_store(i, b):
            pltpu.make_async_copy(rows_bufs[b],
                                  out_hbm.at[pl.ds((base + i * CHUNK) * d, CHUNK * d)],
                                  osem[b]).wait()

        assemble(0, 0)
        start_store(0, 0)
        assemble(1, 1)
        start_store(1, 1)

        def body(j, carry):
            i0 = 2 * j
            i1 = i0 + 1
            wait_store(i0 - 2, 0)
            assemble(i0, 0)
            start_store(i0, 0)
            wait_store(i1 - 2, 1)
            assemble(i1, 1)
            start_store(i1, 1)
            return carry

        lax.fori_loop(1, n_chunks // 2, body, 0)

        wait_store(n_chunks - 2, 0)
        wait_store(n_chunks - 1, 1)

    return lookup


def kernel(idx, x, table):
    del x
    b, l = idx.shape
    n = b * l
    d = table.shape[1]
    idx_flat = idx.reshape(n).astype(jnp.int32)
    table_flat = table.astype(jnp.float32).reshape(-1)
    lookup = _make_lookup(n, table.shape[0], d)
    out = lookup(table_flat, idx_flat)
    return out.reshape(b, l, d)


# speedup vs baseline: 14.2122x; 2.9602x over previous
import functools

import jax
import jax.numpy as jnp
from jax import lax
from jax.experimental import pallas as pl
from jax.experimental.pallas import tpu as pltpu
from jax.experimental.pallas import tpu_sc as plsc

EMB_DIM = 128
CHUNK = 128


@functools.lru_cache(maxsize=None)
def _make_lookup(n_idx: int, n_emb: int, d: int):
    info = plsc.get_sparse_core_info()
    nw = info.num_cores * info.num_subcores
    assert n_idx % (nw * 2 * CHUNK) == 0
    per_w = n_idx // nw
    n_chunks = per_w // CHUNK
    mesh = plsc.VectorSubcoreMesh(core_axis_name="c", subcore_axis_name="s")

    @functools.partial(
        pl.kernel,
        mesh=mesh,
        out_type=jax.ShapeDtypeStruct((n_idx, d), jnp.float32),
        scratch_types=[
            pltpu.VMEM((n_emb, d), jnp.float32),
            pltpu.VMEM_SHARED((n_emb, d), jnp.float32),
            pltpu.VMEM((2, CHUNK), jnp.int32),
            pltpu.VMEM((2, CHUNK, d), jnp.float32),
            pltpu.SemaphoreType.DMA,
            pltpu.SemaphoreType.DMA,
            pltpu.SemaphoreType.DMA,
            pltpu.SemaphoreType.DMA,
        ],
    )
    def lookup(table_hbm, idx_hbm, out_hbm, table_v, table_sp, idx_v, rows_v,
               g0, g1, o0, o1):
        wid = lax.axis_index("s") * info.num_cores + lax.axis_index("c")
        base = wid * per_w
        gsem = (g0, g1)
        osem = (o0, o1)

        @pl.when(lax.axis_index("s") == 0)
        def _():
            pltpu.sync_copy(table_hbm, table_v)
            pltpu.sync_copy(table_v, table_sp)

        plsc.subcore_barrier()

        def start_gather(i, b):
            pltpu.sync_copy(idx_hbm.at[pl.ds(base + i * CHUNK, CHUNK)],
                            idx_v.at[b])
            pltpu.async_copy(table_sp.at[idx_v.at[b]], rows_v.at[b], gsem[b])

        def wait_gather(b):
            pltpu.make_async_copy(table_sp.at[idx_v.at[b]], rows_v.at[b],
                                  gsem[b]).wait()

        def start_store(i, b):
            pltpu.async_copy(rows_v.at[b],
                             out_hbm.at[pl.ds(base + i * CHUNK, CHUNK)],
                             osem[b])

        def wait_store(i, b):
            pltpu.make_async_copy(rows_v.at[b],
                                  out_hbm.at[pl.ds(base + i * CHUNK, CHUNK)],
                                  osem[b]).wait()

        start_gather(0, 0)
        start_gather(1, 1)
        wait_gather(0)
        start_store(0, 0)

        def body(j, carry):
            i0 = 2 * j
            i1 = i0 + 1
            wait_store(i0 - 2, 0)
            start_gather(i0, 0)
            wait_gather(1)
            start_store(i0 - 1, 1)
            wait_store(i1 - 2, 1)
            start_gather(i1, 1)
            wait_gather(0)
            start_store(i0, 0)
            return carry

        lax.fori_loop(1, n_chunks // 2, body, 0)

        wait_gather(1)
        start_store(n_chunks - 1, 1)
        wait_store(n_chunks - 2, 0)
        wait_store(n_chunks - 1, 1)

    return lookup


def kernel(idx, x, table):
    del x
    b, l = idx.shape
    n = b * l
    idx_flat = idx.reshape(n).astype(jnp.int32)
    lookup = _make_lookup(n, table.shape[0], table.shape[1])
    out = lookup(table.astype(jnp.float32), idx_flat)
    return out.reshape(b, l, table.shape[1])


# speedup vs baseline: 17.8311x; 1.2546x over previous
import functools

import jax
import jax.numpy as jnp
from jax import lax
from jax.experimental import pallas as pl
from jax.experimental.pallas import tpu as pltpu
from jax.experimental.pallas import tpu_sc as plsc

EMB_DIM = 128
G = 128
NG = 2
SUP = G * NG


@functools.lru_cache(maxsize=None)
def _make_lookup(n_idx: int, n_emb: int, d: int):
    info = plsc.get_sparse_core_info()
    nw = info.num_cores * info.num_subcores
    assert n_idx % (nw * 2 * SUP) == 0
    per_w = n_idx // nw
    n_chunks = per_w // SUP
    mesh = plsc.VectorSubcoreMesh(core_axis_name="c", subcore_axis_name="s")

    @functools.partial(
        pl.kernel,
        mesh=mesh,
        out_type=jax.ShapeDtypeStruct((n_idx, d), jnp.float32),
        scratch_types=[
            pltpu.VMEM((n_emb, d), jnp.float32),
            pltpu.VMEM_SHARED((n_emb, d), jnp.float32),
            pltpu.VMEM((2, SUP), jnp.int32),
            pltpu.VMEM((2, SUP, d), jnp.float32),
            pltpu.SemaphoreType.DMA,
            pltpu.SemaphoreType.DMA,
            pltpu.SemaphoreType.DMA,
            pltpu.SemaphoreType.DMA,
        ],
    )
    def lookup(table_hbm, idx_hbm, out_hbm, table_v, table_sp, idx_v, rows_v,
               g0, g1, o0, o1):
        wid = lax.axis_index("s") * info.num_cores + lax.axis_index("c")
        base = wid * per_w
        gsem = (g0, g1)
        osem = (o0, o1)

        @pl.when(lax.axis_index("s") == 0)
        def _():
            pltpu.sync_copy(table_hbm, table_v)
            pltpu.sync_copy(table_v, table_sp)

        plsc.subcore_barrier()

        def start_gather(i, b):
            pltpu.sync_copy(idx_hbm.at[pl.ds(base + i * SUP, SUP)],
                            idx_v.at[b])
            for k in range(NG):
                pltpu.async_copy(table_sp.at[idx_v.at[b, pl.ds(k * G, G)]],
                                 rows_v.at[b, pl.ds(k * G, G)], gsem[b])

        def wait_gather(b):
            for k in range(NG):
                pltpu.make_async_copy(
                    table_sp.at[idx_v.at[b, pl.ds(k * G, G)]],
                    rows_v.at[b, pl.ds(k * G, G)], gsem[b]).wait()

        def start_store(i, b):
            pltpu.async_copy(rows_v.at[b],
                             out_hbm.at[pl.ds(base + i * SUP, SUP)],
                             osem[b])

        def wait_store(i, b):
            pltpu.make_async_copy(rows_v.at[b],
                                  out_hbm.at[pl.ds(base + i * SUP, SUP)],
                                  osem[b]).wait()

        start_gather(0, 0)
        start_gather(1, 1)
        wait_gather(0)
        start_store(0, 0)

        def body(j, carry):
            i0 = 2 * j
            i1 = i0 + 1
            wait_store(i0 - 2, 0)
            start_gather(i0, 0)
            wait_gather(1)
            start_store(i0 - 1, 1)
            wait_store(i1 - 2, 1)
            start_gather(i1, 1)
            wait_gather(0)
            start_store(i0, 0)
            return carry

        lax.fori_loop(1, n_chunks // 2, body, 0)

        wait_gather(1)
        start_store(n_chunks - 1, 1)
        wait_store(n_chunks - 2, 0)
        wait_store(n_chunks - 1, 1)

    return lookup


def kernel(idx, x, table):
    del x
    b, l = idx.shape
    n = b * l
    idx_flat = idx.reshape(n).astype(jnp.int32)
    lookup = _make_lookup(n, table.shape[0], table.shape[1])
    out = lookup(table.astype(jnp.float32), idx_flat)
    return out.reshape(b, l, table.shape[1])


# speedup vs baseline: 22.2923x; 1.2502x over previous
import functools

import jax
import jax.numpy as jnp
from jax import lax
from jax.experimental import pallas as pl
from jax.experimental.pallas import tpu as pltpu
from jax.experimental.pallas import tpu_sc as plsc

EMB_DIM = 128
G = 128
NG = 2
SUP = G * NG


@functools.lru_cache(maxsize=None)
def _make_lookup(n_idx: int, n_emb: int, d: int):
    info = plsc.get_sparse_core_info()
    nw = info.num_cores * info.num_subcores
    assert n_idx % (nw * 2 * SUP) == 0
    per_w = n_idx // nw
    n_chunks = per_w // SUP
    mesh = plsc.VectorSubcoreMesh(core_axis_name="c", subcore_axis_name="s")

    @functools.partial(
        pl.kernel,
        mesh=mesh,
        out_type=jax.ShapeDtypeStruct((n_idx, d), jnp.float32),
        scratch_types=[
            pltpu.VMEM((n_emb, d), jnp.float32),
            pltpu.VMEM_SHARED((n_emb, d), jnp.float32),
            pltpu.VMEM((2, SUP), jnp.int32),
            pltpu.VMEM((2, SUP, d), jnp.float32),
            pltpu.SemaphoreType.DMA,
            pltpu.SemaphoreType.DMA,
            pltpu.SemaphoreType.DMA,
            pltpu.SemaphoreType.DMA,
            pltpu.SemaphoreType.DMA,
            pltpu.SemaphoreType.DMA,
        ],
    )
    def lookup(table_hbm, idx_hbm, out_hbm, table_v, table_sp, idx_v, rows_v,
               g0, g1, o0, o1, i0sem, i1sem):
        wid = lax.axis_index("s") * info.num_cores + lax.axis_index("c")
        base = wid * per_w
        gsem = (g0, g1)
        osem = (o0, o1)
        isem = (i0sem, i1sem)

        @pl.when(lax.axis_index("s") == 0)
        def _():
            pltpu.sync_copy(table_hbm, table_v)
            pltpu.sync_copy(table_v, table_sp)

        plsc.subcore_barrier()

        def start_idx(i, b):
            iw = lax.rem(i, n_chunks)
            pltpu.async_copy(idx_hbm.at[pl.ds(base + iw * SUP, SUP)],
                             idx_v.at[b], isem[b])

        def wait_idx(i, b):
            iw = lax.rem(i, n_chunks)
            pltpu.make_async_copy(idx_hbm.at[pl.ds(base + iw * SUP, SUP)],
                                  idx_v.at[b], isem[b]).wait()

        def start_gather(i, b):
            for k in range(NG):
                pltpu.async_copy(table_sp.at[idx_v.at[b, pl.ds(k * G, G)]],
                                 rows_v.at[b, pl.ds(k * G, G)], gsem[b])

        def wait_gather(b):
            for k in range(NG):
                pltpu.make_async_copy(
                    table_sp.at[idx_v.at[b, pl.ds(k * G, G)]],
                    rows_v.at[b, pl.ds(k * G, G)], gsem[b]).wait()

        def start_store(i, b):
            pltpu.async_copy(rows_v.at[b],
                             out_hbm.at[pl.ds(base + i * SUP, SUP)],
                             osem[b])

        def wait_store(i, b):
            pltpu.make_async_copy(rows_v.at[b],
                                  out_hbm.at[pl.ds(base + i * SUP, SUP)],
                                  osem[b]).wait()

        start_idx(0, 0)
        wait_idx(0, 0)
        start_gather(0, 0)
        start_idx(1, 1)
        wait_idx(1, 1)
        start_gather(1, 1)
        wait_gather(0)
        start_store(0, 0)
        start_idx(2, 0)

        def body(j, carry):
            i0 = 2 * j
            i1 = i0 + 1
            wait_store(i0 - 2, 0)
            wait_idx(i0, 0)
            start_gather(i0, 0)
            wait_gather(1)
            start_store(i0 - 1, 1)
            start_idx(i1, 1)
            wait_store(i1 - 2, 1)
            wait_idx(i1, 1)
            start_gather(i1, 1)
            wait_gather(0)
            start_store(i0, 0)
            start_idx(i0 + 2, 0)
            return carry

        lax.fori_loop(1, n_chunks // 2, body, 0)

        wait_gather(1)
        start_store(n_chunks - 1, 1)
        wait_idx(n_chunks, 0)
        wait_store(n_chunks - 2, 0)
        wait_store(n_chunks - 1, 1)

    return lookup


def kernel(idx, x, table):
    del x
    b, l = idx.shape
    n = b * l
    idx_flat = idx.reshape(n).astype(jnp.int32)
    lookup = _make_lookup(n, table.shape[0], table.shape[1])
    out = lookup(table.astype(jnp.float32), idx_flat)
    return out.reshape(b, l, table.shape[1])
